# Initial kernel scaffold; baseline (speedup 1.0000x reference)
import jax
import jax.numpy as jnp
from jax.experimental import pallas as pl


def kernel(l2c_w0, l2c_b0, l2c_w1, l2c_b1, c2l_w0, c2l_b0, c2l_w1, c2l_b1, l2l_w0, l2l_b0, l2l_w1, l2l_b1, c_update_w, c_update_b, l_update_w, l_update_b, l_edge_index, c_edge_index, l_emb, c_emb):
    raise NotImplementedError("write your pallas kernel here")


# speedup vs baseline: 1.4302x; 1.4302x over previous
from functools import partial

import jax
import jax.numpy as jnp
from jax import lax
from jax.experimental import pallas as pl
from jax.experimental.pallas import tpu as pltpu

F32 = jnp.float32
_VMEM_LIM = 100 * 1024 * 1024


def _msg_body(x_ref, s_ref, w0, b0, w1, b1, wu, o_ref):
    x = x_ref[...]
    h = jnp.maximum(jnp.dot(x, w0[...], preferred_element_type=F32) + b0[...], 0.0)
    m = jnp.dot(h, w1[...], preferred_element_type=F32) + b1[...]
    m = m * s_ref[...]
    o_ref[...] = jnp.dot(m, wu[...], preferred_element_type=F32)


def _messages(x, s, w0, b0, w1, b1, wu, tm):
    rows, d = x.shape
    n = rows // tm
    wspec = pl.BlockSpec((d, d), lambda i: (0, 0))
    bspec = pl.BlockSpec((1, d), lambda i: (0, 0))
    return pl.pallas_call(
        _msg_body,
        out_shape=jax.ShapeDtypeStruct((rows, d), F32),
        grid=(n,),
        in_specs=[pl.BlockSpec((tm, d), lambda i: (i, 0)),
                  pl.BlockSpec((tm, 1), lambda i: (i, 0)),
                  wspec, bspec, wspec, bspec, wspec],
        out_specs=pl.BlockSpec((tm, d), lambda i: (i, 0)),
        compiler_params=pltpu.CompilerParams(
            dimension_semantics=("parallel",),
            vmem_limit_bytes=_VMEM_LIM),
    )(x, s, w0, b0.reshape(1, d), w1, b1.reshape(1, d), wu)


def _scatter_body(e_per_core, n_rep, src_ref, dst_ref, y_ref, out_ref, *reps):
    g = pl.program_id(0)
    base = g * e_per_core
    accs = (out_ref,) + reps

    out_ref[...] = jnp.zeros_like(out_ref)
    for r in reps:
        r[...] = jnp.zeros_like(r)

    def body(b, carry):
        e0 = base + b * n_rep
        vals = []
        for u in range(n_rep):
            s = src_ref[e0 + u]
            d = dst_ref[e0 + u]
            vals.append((d, accs[u][d, 0] + y_ref[s, 0]))
        for u in range(n_rep):
            d, v = vals[u]
            accs[u][d, 0] = v
        return carry

    lax.fori_loop(0, e_per_core // n_rep, body, 0)

    tot = out_ref[...]
    for r in reps:
        tot = tot + r[...]
    out_ref[...] = tot


def _scatter(y, src, dst, n_dst, n_rep):
    n_src, d = y.shape
    n_edges = src.shape[0]
    e_per_core = n_edges // 2
    out = pl.pallas_call(
        partial(_scatter_body, e_per_core, n_rep),
        out_shape=jax.ShapeDtypeStruct((2 * n_dst, 1, d), F32),
        grid_spec=pltpu.PrefetchScalarGridSpec(
            num_scalar_prefetch=2,
            grid=(2,),
            in_specs=[pl.BlockSpec((n_src, 1, d), lambda g, s_, d_: (0, 0, 0))],
            out_specs=pl.BlockSpec((n_dst, 1, d), lambda g, s_, d_: (g, 0, 0)),
            scratch_shapes=[pltpu.VMEM((n_dst, 1, d), F32)] * (n_rep - 1)),
        compiler_params=pltpu.CompilerParams(
            dimension_semantics=("parallel",),
            vmem_limit_bytes=_VMEM_LIM),
    )(src, dst, y.reshape(n_src, 1, d))
    return out.reshape(2 * n_dst, d)


def _updc_body(x_ref, p0_ref, p1_ref, s_ref, w_ref, b_ref, o_ref):
    agg = (p0_ref[...] + p1_ref[...]) * s_ref[...]
    o_ref[...] = (jnp.dot(x_ref[...], w_ref[...], preferred_element_type=F32)
                  + agg + b_ref[...])


def _update_c(c_emb, parts, s, w, b, tm):
    rows, d = c_emb.shape
    n = rows // tm
    off = rows // tm
    return pl.pallas_call(
        _updc_body,
        out_shape=jax.ShapeDtypeStruct((rows, d), F32),
        grid=(n,),
        in_specs=[pl.BlockSpec((tm, d), lambda i: (i, 0)),
                  pl.BlockSpec((tm, d), lambda i: (i, 0)),
                  pl.BlockSpec((tm, d), lambda i: (i + off, 0)),
                  pl.BlockSpec((tm, 1), lambda i: (i, 0)),
                  pl.BlockSpec((d, d), lambda i: (0, 0)),
                  pl.BlockSpec((1, d), lambda i: (0, 0))],
        out_specs=pl.BlockSpec((tm, d), lambda i: (i, 0)),
        compiler_params=pltpu.CompilerParams(
            dimension_semantics=("parallel",),
            vmem_limit_bytes=_VMEM_LIM),
    )(c_emb, parts, parts, s, w, b.reshape(1, d))


def _updl_body(x_ref, p0_ref, p1_ref, s_ref, wl0, bl,
               w0, b0, w1, b1, w2, o_ref):
    x = x_ref[...]
    rows = lax.broadcasted_iota(jnp.int32, x.shape, 0)
    up = pltpu.roll(x, x.shape[0] - 1, axis=0)
    dn = pltpu.roll(x, 1, axis=0)
    xs = jnp.where((rows % 2) == 0, up, dn)
    h = jnp.maximum(jnp.dot(xs, w0[...], preferred_element_type=F32) + b0[...], 0.0)
    m = jnp.dot(h, w1[...], preferred_element_type=F32) + b1[...]
    y2 = jnp.dot(m, w2[...], preferred_element_type=F32)
    agg = (p0_ref[...] + p1_ref[...]) * s_ref[...]
    o_ref[...] = (jnp.dot(x, wl0[...], preferred_element_type=F32)
                  + agg + y2 + bl[...])


def _update_l(l_emb, parts, s, wl0, bl, w0, b0, w1, b1, w2, tm):
    rows, d = l_emb.shape
    n = rows // tm
    off = rows // tm
    wspec = pl.BlockSpec((d, d), lambda i: (0, 0))
    bspec = pl.BlockSpec((1, d), lambda i: (0, 0))
    return pl.pallas_call(
        _updl_body,
        out_shape=jax.ShapeDtypeStruct((rows, d), F32),
        grid=(n,),
        in_specs=[pl.BlockSpec((tm, d), lambda i: (i, 0)),
                  pl.BlockSpec((tm, d), lambda i: (i, 0)),
                  pl.BlockSpec((tm, d), lambda i: (i + off, 0)),
                  pl.BlockSpec((tm, 1), lambda i: (i, 0)),
                  wspec, bspec, wspec, bspec, wspec, bspec, wspec],
        out_specs=pl.BlockSpec((tm, d), lambda i: (i, 0)),
        compiler_params=pltpu.CompilerParams(
            dimension_semantics=("parallel",),
            vmem_limit_bytes=_VMEM_LIM),
    )(l_emb, parts, parts, s, wl0, bl.reshape(1, d),
      w0, b0.reshape(1, d), w1, b1.reshape(1, d), w2)


def kernel(l2c_w0, l2c_b0, l2c_w1, l2c_b1,
           c2l_w0, c2l_b0, c2l_w1, c2l_b1,
           l2l_w0, l2l_b0, l2l_w1, l2l_b1,
           c_update_w, c_update_b, l_update_w, l_update_b,
           l_edge_index, c_edge_index, l_emb, c_emb):
    L, D = l_emb.shape
    C = c_emb.shape[0]
    E = l_edge_index.shape[0]
    N_ITERATIONS = 2
    TM_L = min(4096, L)
    TM_C = min(2048, C)

    le = l_edge_index.astype(jnp.int32)
    ce = c_edge_index.astype(jnp.int32)

    ones_e = jnp.ones((E,), F32)
    l_deg = jax.ops.segment_sum(ones_e, le, num_segments=L)
    c_deg = jax.ops.segment_sum(ones_e, ce, num_segments=C)
    inv_l = jnp.where(l_deg > 0, lax.rsqrt(l_deg), 0.0).reshape(L, 1)
    inv_c = jnp.where(c_deg > 0, lax.rsqrt(c_deg), 0.0).reshape(C, 1)

    wc0, wc1 = c_update_w[:D], c_update_w[D:]
    wl0, wl1, wl2 = l_update_w[:D], l_update_w[D:2 * D], l_update_w[2 * D:]

    l_embs, c_embs = [l_emb], [c_emb]
    for _ in range(N_ITERATIONS):
        y_l = _messages(l_emb, inv_l, l2c_w0, l2c_b0, l2c_w1, l2c_b1, wc1, TM_L)
        y_c = _messages(c_emb, inv_c, c2l_w0, c2l_b0, c2l_w1, c2l_b1, wl1, TM_C)

        parts_c = _scatter(y_l, le, ce, C, n_rep=4)
        c_new = _update_c(c_emb, parts_c, inv_c, wc0, c_update_b, TM_C)

        parts_l = _scatter(y_c, ce, le, L, n_rep=2)
        l_new = _update_l(l_emb, parts_l, inv_l, wl0, l_update_b,
                          l2l_w0, l2l_b0, l2l_w1, l2l_b1, wl2, TM_L)

        l_emb, c_emb = l_new, c_new
        l_embs.append(l_emb)
        c_embs.append(c_emb)

    return l_embs, c_embs
